# baseline (speedup 1.0000x reference)
import functools

import jax
import jax.numpy as jnp
from jax import lax
from jax.experimental import pallas as pl
from jax.experimental.pallas import tpu as pltpu
from jax.experimental.pallas import tpu_sc as plsc

_NUM_CORES = 2
_NUM_SUBCORES = 16
_NW = _NUM_CORES * _NUM_SUBCORES
_LANES = 16
_CH = 80
_NSLOT = 3


def _mesh():
    return plsc.VectorSubcoreMesh(
        core_axis_name="c",
        subcore_axis_name="s",
        num_cores=_NUM_CORES,
        num_subcores=_NUM_SUBCORES,
    )


@functools.partial(jax.jit, static_argnames=("bb", "ll", "zrow"))
def _sc_mask(idx_flat, slen, bb, ll, zrow):
    n_rows = bb * ll
    rpw = n_rows // _NW
    spw = bb // _NW

    @functools.partial(
        pl.kernel,
        out_type=jax.ShapeDtypeStruct((n_rows,), jnp.int32),
        mesh=_mesh(),
        compiler_params=pltpu.CompilerParams(use_tc_tiling_on_sc=False),
        scratch_types=[
            pltpu.VMEM((rpw,), jnp.int32),
            pltpu.VMEM((spw + _LANES,), jnp.int32),
        ],
    )
    def run(idx_hbm, slen_hbm, out_hbm, idxv, slen_v):
        wid = lax.axis_index("s") * _NUM_CORES + lax.axis_index("c")
        base = wid * rpw
        pltpu.sync_copy(
            slen_hbm.at[pl.ds(wid * spw, spw)], slen_v.at[pl.ds(0, spw)]
        )
        pltpu.sync_copy(idx_hbm.at[pl.ds(base, rpw)], idxv)

        lane = lax.iota(jnp.int32, _LANES)
        zv = jnp.full((_LANES,), zrow, jnp.int32)
        offs = [j * _LANES for j in range(ll // _LANES)]
        if ll % _LANES:
            offs.append(ll - _LANES)

        def mask_seq(i, carry):
            n = slen_v[pl.ds(i, _LANES)][0]
            nv = jnp.full((_LANES,), n, jnp.int32)
            b = i * ll
            for off in offs:
                iv = idxv[pl.ds(b + off, _LANES)]
                idxv[pl.ds(b + off, _LANES)] = jnp.where(
                    lane + off < nv, iv, zv
                )
            return carry

        lax.fori_loop(0, spw, mask_seq, 0)
        pltpu.sync_copy(idxv, out_hbm.at[pl.ds(base, rpw)])

    return run(idx_flat, slen)


@functools.partial(jax.jit, static_argnames=("dp",))
def _sc_gather(midx, table_p, dp):
    n_rows = midx.shape[0]
    rpw = n_rows // _NW
    cpw = rpw // _CH

    @functools.partial(
        pl.kernel,
        out_type=jax.ShapeDtypeStruct((n_rows, dp), jnp.float32),
        mesh=_mesh(),
        compiler_params=pltpu.CompilerParams(use_tc_tiling_on_sc=False),
        scratch_types=[
            pltpu.VMEM((rpw + _LANES,), jnp.int32),
            pltpu.VMEM((2, _CH, dp), jnp.float32),
            [pltpu.SemaphoreType.DMA] * 2,
        ],
    )
    def run(midx_hbm, table_hbm, out_hbm, idxv, bufs, gsems):
        wid = lax.axis_index("s") * _NUM_CORES + lax.axis_index("c")
        base = wid * rpw
        pltpu.sync_copy(midx_hbm.at[pl.ds(base, rpw)], idxv.at[pl.ds(0, rpw)])

        def g_start(c, slot):
            for k in range(_CH):
                r = idxv[pl.ds(c * _CH + k, _LANES)][0]
                pltpu.async_copy(
                    table_hbm.at[r], bufs.at[slot, k], gsems[slot]
                )

        def g_wait(slot):
            pltpu.make_async_copy(
                out_hbm.at[pl.ds(0, _CH)], bufs.at[slot], gsems[slot]
            ).wait()

        def w_sync(c, slot):
            pltpu.sync_copy(
                bufs.at[slot], out_hbm.at[pl.ds(base + c * _CH, _CH)]
            )

        g_start(0, 0)
        g_start(1, 1)

        def step(p, carry):
            c0 = 2 * p
            g_wait(0)
            w_sync(c0, 0)
            g_start(c0 + 2, 0)
            g_wait(1)
            w_sync(c0 + 1, 1)
            g_start(c0 + 3, 1)
            return carry

        lax.fori_loop(0, cpw // 2 - 1, step, 0)

        g_wait(0)
        w_sync(cpw - 2, 0)
        g_wait(1)
        w_sync(cpw - 1, 1)

    return run(midx, table_p)


def kernel(indices, seq_lengths, table):
    bb, ll = indices.shape
    vv, dd = table.shape
    dp = (dd + _LANES - 1) // _LANES * _LANES
    idx_flat = indices.reshape(bb * ll).astype(jnp.int32)
    slen = seq_lengths.astype(jnp.int32)
    table_p = jnp.pad(table.astype(jnp.float32), ((0, 8), (0, dp - dd)))
    midx = _sc_mask(idx_flat, slen, bb, ll, vv)
    out = _sc_gather(midx, table_p, dp)
    return out[:, :dd].reshape(bb, ll, dd)
